# baseline (speedup 1.0000x reference)
import jax
import jax.numpy as jnp
from jax import lax
from jax.experimental import pallas as pl
from jax.experimental.pallas import tpu as pltpu
from jax.experimental.pallas import tpu_sc as plsc

N = 32768
B = 16
D = 768
L = 16
NC, NS = 2, 16
NW = NC * NS
S_SC = 2048
S_TC = N - S_SC
ROWS_W = S_SC // NW
CH = 32
NCH = ROWS_W // CH
NV = D // L
BLOCK_R = 2048


def _sc_body(xf_hbm, batch_hbm, w_hbm, bvec_hbm,
             wout_hbm, part_hbm, cnt_hbm,
             xb0, xb1, batchb, wv, bv, woutb, accb, cntb,
             sem0, sem1):
    c = lax.axis_index("c")
    s = lax.axis_index("s")
    wid = s * NC + c
    row0 = S_TC + wid * ROWS_W

    pltpu.sync_copy(w_hbm, wv)
    pltpu.sync_copy(bvec_hbm, bv)
    pltpu.sync_copy(batch_hbm.at[pl.ds(row0, ROWS_W)], batchb)

    zero16 = jnp.zeros((L,), jnp.float32)

    def zb_(j, carry):
        accb[pl.ds(j * L, L)] = zero16
        return carry

    lax.fori_loop(0, B * D // L, zb_, 0)

    pltpu.async_copy(xf_hbm.at[pl.ds(row0, CH)], xb0, sem0)

    iot = lax.broadcasted_iota(jnp.int32, (L,), 0)

    def lane_gather(v, idx):
        return lax.gather(
            v, idx.reshape(L, 1),
            lax.GatherDimensionNumbers(
                offset_dims=(), collapsed_slice_dims=(0,), start_index_map=(0,)),
            (1,), mode=lax.GatherScatterMode.PROMISE_IN_BOUNDS)

    def process(xb, crb, cnt_vec):
        def row_pair_body(p, cnt_vec):
            rows = [2 * p, 2 * p + 1]
            accp = [[zero16, zero16, zero16, zero16] for _ in range(2)]
            for j in range(NV):
                for q in range(2):
                    accp[q][j % 4] = accp[q][j % 4] + (
                        xb[rows[q], pl.ds(j * L, L)] * wv[pl.ds(j * L, L)])
            zbs = []
            for q in range(2):
                a = accp[q]
                accv = (a[0] + a[1]) + (a[2] + a[3])
                for sh in (8, 4, 2, 1):
                    accv = accv + lane_gather(accv, (iot + sh) & (L - 1))
                zbs.append(accv + bv[...])
            w_rows, bases = [], []
            for q in range(2):
                zb = zbs[q]
                w_row = 1.0 / (1.0 + jnp.exp(-zb))
                cnt_vec = cnt_vec + jnp.where(zb >= 0.0, 1.0 / L, 0.0)
                rid = jnp.full((L,), crb + 2 * p + q, jnp.int32)
                plsc.store_scatter(woutb, [rid], w_row, mask=iot == 0)
                seg_b = plsc.load_gather(batchb, [rid])
                w_rows.append(w_row)
                bases.append(seg_b * D + iot)
            for j in range(NV):
                for q in range(2):
                    v = xb[rows[q], pl.ds(j * L, L)] * w_rows[q]
                    plsc.addupdate_scatter(accb, [bases[q] + j * L], v)
            return cnt_vec

        return lax.fori_loop(0, CH // 2, row_pair_body, cnt_vec)

    def pair(i, cnt_vec):
        off1 = (2 * i + 1) * CH
        pltpu.async_copy(xf_hbm.at[pl.ds(row0 + off1, CH)], xb1, sem1)
        pltpu.make_async_copy(xf_hbm.at[pl.ds(0, CH)], xb0, sem0).wait()
        cnt_vec = process(xb0, (2 * i) * CH, cnt_vec)

        @pl.when(i < NCH // 2 - 1)
        def _():
            off2 = (2 * i + 2) * CH
            pltpu.async_copy(xf_hbm.at[pl.ds(row0 + off2, CH)], xb0, sem0)

        pltpu.make_async_copy(xf_hbm.at[pl.ds(0, CH)], xb1, sem1).wait()
        cnt_vec = process(xb1, off1, cnt_vec)
        return cnt_vec

    cnt_vec = lax.fori_loop(0, NCH // 2, pair, zero16)

    cntb[...] = cnt_vec
    pltpu.sync_copy(woutb, wout_hbm.at[pl.ds(row0 - S_TC, ROWS_W)])
    pltpu.sync_copy(accb, part_hbm.at[wid])
    pltpu.sync_copy(cntb, cnt_hbm.at[wid])


def _tc_body(x_ref, batch_ref, w_ref, b_ref, pooled_ref, weights_ref, cnt_ref):
    i = pl.program_id(0)

    xb = x_ref[...]
    z = lax.dot_general(
        xb, w_ref[...], (((1,), (0,)), ((), ())),
        preferred_element_type=jnp.float32,
    ) + b_ref[0, 0]
    w = jax.nn.sigmoid(z)
    weights_ref[...] = w
    xw = xb * w

    seg = batch_ref[...]
    onehot = (seg == lax.broadcasted_iota(jnp.int32, (BLOCK_R, B), 1)
              ).astype(jnp.float32)
    partial = lax.dot_general(
        onehot, xw, (((0,), (0,)), ((), ())),
        preferred_element_type=jnp.float32,
    )
    cnt = jnp.sum((z >= 0.0).astype(jnp.float32)).reshape(1, 1)

    @pl.when(i == 0)
    def _init():
        pooled_ref[...] = jnp.zeros_like(pooled_ref)
        cnt_ref[...] = jnp.zeros((1, 1), jnp.float32)

    pooled_ref[...] += partial
    cnt_ref[...] += cnt


def _combine_body(part_ref, cnt_ref, tcpool_ref, tccnt_ref, pooled_ref, ratio_ref):
    p = part_ref[...]
    pooled_ref[...] = tcpool_ref[...] + jnp.sum(p, axis=0)
    total = jnp.sum(cnt_ref[...]) + tccnt_ref[0, 0]
    ratio_ref[...] = total.reshape(1, 1) * (1.0 / N)


def kernel(x, batch, ptr, W, b):
    del ptr
    wf = W.reshape(-1)
    bvec = jnp.broadcast_to(b, (L,))
    batch2 = batch.reshape(N, 1)
    b2 = b.reshape(1, 1)

    mesh = plsc.VectorSubcoreMesh(core_axis_name="c", subcore_axis_name="s",
                                  num_cores=NC, num_subcores=NS)
    wflat_sc, part, cnt_sc = pl.kernel(
        _sc_body,
        out_type=[
            jax.ShapeDtypeStruct((S_SC,), jnp.float32),
            jax.ShapeDtypeStruct((NW, B * D), jnp.float32),
            jax.ShapeDtypeStruct((NW, L), jnp.float32),
        ],
        mesh=mesh,
        compiler_params=pltpu.CompilerParams(needs_layout_passes=False),
        scratch_types=[
            pltpu.VMEM((CH, D), jnp.float32),
            pltpu.VMEM((CH, D), jnp.float32),
            pltpu.VMEM((ROWS_W,), jnp.int32),
            pltpu.VMEM((D,), jnp.float32),
            pltpu.VMEM((L,), jnp.float32),
            pltpu.VMEM((ROWS_W,), jnp.float32),
            pltpu.VMEM((B * D,), jnp.float32),
            pltpu.VMEM((L,), jnp.float32),
            pltpu.SemaphoreType.DMA,
            pltpu.SemaphoreType.DMA,
        ],
    )(x, batch, wf, bvec)

    pooled_tc, weights_tc, cnt_tc = pl.pallas_call(
        _tc_body,
        grid=(S_TC // BLOCK_R,),
        in_specs=[
            pl.BlockSpec((BLOCK_R, D), lambda i: (i, 0)),
            pl.BlockSpec((BLOCK_R, 1), lambda i: (i, 0)),
            pl.BlockSpec((D, 1), lambda i: (0, 0)),
            pl.BlockSpec((1, 1), lambda i: (0, 0)),
        ],
        out_specs=[
            pl.BlockSpec((B, D), lambda i: (0, 0)),
            pl.BlockSpec((BLOCK_R, 1), lambda i: (i, 0)),
            pl.BlockSpec((1, 1), lambda i: (0, 0)),
        ],
        out_shape=[
            jax.ShapeDtypeStruct((B, D), jnp.float32),
            jax.ShapeDtypeStruct((S_TC, 1), jnp.float32),
            jax.ShapeDtypeStruct((1, 1), jnp.float32),
        ],
        compiler_params=pltpu.CompilerParams(
            dimension_semantics=("arbitrary",),
        ),
    )(x, batch2, W, b2)

    pooled, ratio = pl.pallas_call(
        _combine_body,
        out_shape=[
            jax.ShapeDtypeStruct((B, D), jnp.float32),
            jax.ShapeDtypeStruct((1, 1), jnp.float32),
        ],
    )(part.reshape(NW, B, D), cnt_sc, pooled_tc, cnt_tc)

    weights = jnp.concatenate([weights_tc, wflat_sc.reshape(S_SC, 1)], axis=0)
    return pooled, weights, ratio.reshape(())


# speedup vs baseline: 1.0661x; 1.0661x over previous
import jax
import jax.numpy as jnp
from jax import lax
from jax.experimental import pallas as pl
from jax.experimental.pallas import tpu as pltpu
from jax.experimental.pallas import tpu_sc as plsc

N = 32768
B = 16
D = 768
L = 16
NC, NS = 2, 16
NW = NC * NS
S_SC = 4096
S_TC = N - S_SC
ROWS_W = S_SC // NW
CH = 32
NCH = ROWS_W // CH
NV = D // L
BLOCK_R = 4096


def _sc_body(xf_hbm, batch_hbm, w_hbm, bvec_hbm,
             wout_hbm, part_hbm, cnt_hbm,
             xb0, xb1, batchb, wv, bv, woutb, accb, cntb,
             sem0, sem1):
    c = lax.axis_index("c")
    s = lax.axis_index("s")
    wid = s * NC + c
    row0 = S_TC + wid * ROWS_W

    pltpu.sync_copy(w_hbm, wv)
    pltpu.sync_copy(bvec_hbm, bv)
    pltpu.sync_copy(batch_hbm.at[pl.ds(row0, ROWS_W)], batchb)

    zero16 = jnp.zeros((L,), jnp.float32)

    def zb_(j, carry):
        accb[pl.ds(j * L, L)] = zero16
        return carry

    lax.fori_loop(0, B * D // L, zb_, 0)

    pltpu.async_copy(xf_hbm.at[pl.ds(row0, CH)], xb0, sem0)

    iot = lax.broadcasted_iota(jnp.int32, (L,), 0)

    def lane_gather(v, idx):
        return lax.gather(
            v, idx.reshape(L, 1),
            lax.GatherDimensionNumbers(
                offset_dims=(), collapsed_slice_dims=(0,), start_index_map=(0,)),
            (1,), mode=lax.GatherScatterMode.PROMISE_IN_BOUNDS)

    def process(xb, crb, cnt_vec):
        def row_pair_body(p, cnt_vec):
            rows = [2 * p, 2 * p + 1]
            accp = [[zero16, zero16, zero16, zero16] for _ in range(2)]
            for j in range(NV):
                for q in range(2):
                    accp[q][j % 4] = accp[q][j % 4] + (
                        xb[rows[q], pl.ds(j * L, L)] * wv[pl.ds(j * L, L)])
            zbs = []
            for q in range(2):
                a = accp[q]
                accv = (a[0] + a[1]) + (a[2] + a[3])
                for sh in (8, 4, 2, 1):
                    accv = accv + lane_gather(accv, (iot + sh) & (L - 1))
                zbs.append(accv + bv[...])
            w_rows, bases = [], []
            for q in range(2):
                zb = zbs[q]
                w_row = 1.0 / (1.0 + jnp.exp(-zb))
                cnt_vec = cnt_vec + jnp.where(zb >= 0.0, 1.0 / L, 0.0)
                rid = jnp.full((L,), crb + 2 * p + q, jnp.int32)
                plsc.store_scatter(woutb, [rid], w_row, mask=iot == 0)
                seg_b = plsc.load_gather(batchb, [rid])
                w_rows.append(w_row)
                bases.append(seg_b * D + iot)
            for j in range(NV):
                for q in range(2):
                    v = xb[rows[q], pl.ds(j * L, L)] * w_rows[q]
                    plsc.addupdate_scatter(accb, [bases[q] + j * L], v)
            return cnt_vec

        return lax.fori_loop(0, CH // 2, row_pair_body, cnt_vec)

    def pair(i, cnt_vec):
        off1 = (2 * i + 1) * CH
        pltpu.async_copy(xf_hbm.at[pl.ds(row0 + off1, CH)], xb1, sem1)
        pltpu.make_async_copy(xf_hbm.at[pl.ds(0, CH)], xb0, sem0).wait()
        cnt_vec = process(xb0, (2 * i) * CH, cnt_vec)

        @pl.when(i < NCH // 2 - 1)
        def _():
            off2 = (2 * i + 2) * CH
            pltpu.async_copy(xf_hbm.at[pl.ds(row0 + off2, CH)], xb0, sem0)

        pltpu.make_async_copy(xf_hbm.at[pl.ds(0, CH)], xb1, sem1).wait()
        cnt_vec = process(xb1, off1, cnt_vec)
        return cnt_vec

    cnt_vec = lax.fori_loop(0, NCH // 2, pair, zero16)

    cntb[...] = cnt_vec
    pltpu.sync_copy(woutb, wout_hbm.at[pl.ds(row0 - S_TC, ROWS_W)])
    pltpu.sync_copy(accb, part_hbm.at[wid])
    pltpu.sync_copy(cntb, cnt_hbm.at[wid])


def _tc_body(x_ref, batch_ref, w_ref, b_ref, pooled_ref, weights_ref, cnt_ref):
    i = pl.program_id(0)

    xb = x_ref[...]
    z = lax.dot_general(
        xb, w_ref[...], (((1,), (0,)), ((), ())),
        preferred_element_type=jnp.float32,
    ) + b_ref[0, 0]
    w = jax.nn.sigmoid(z)
    weights_ref[...] = w
    xw = xb * w

    seg = batch_ref[...]
    onehot = (seg == lax.broadcasted_iota(jnp.int32, (BLOCK_R, B), 1)
              ).astype(jnp.float32)
    partial = lax.dot_general(
        onehot, xw, (((0,), (0,)), ((), ())),
        preferred_element_type=jnp.float32,
    )
    cnt = jnp.sum((z >= 0.0).astype(jnp.float32)).reshape(1, 1)

    @pl.when(i == 0)
    def _init():
        pooled_ref[...] = jnp.zeros_like(pooled_ref)
        cnt_ref[...] = jnp.zeros((1, 1), jnp.float32)

    pooled_ref[...] += partial
    cnt_ref[...] += cnt


def _combine_body(part_ref, cnt_ref, tcpool_ref, tccnt_ref, pooled_ref, ratio_ref):
    p = part_ref[...]
    pooled_ref[...] = tcpool_ref[...] + jnp.sum(p, axis=0)
    total = jnp.sum(cnt_ref[...]) + tccnt_ref[0, 0]
    ratio_ref[...] = total.reshape(1, 1) * (1.0 / N)


def kernel(x, batch, ptr, W, b):
    del ptr
    wf = W.reshape(-1)
    bvec = jnp.broadcast_to(b, (L,))
    batch2 = batch.reshape(N, 1)
    b2 = b.reshape(1, 1)

    mesh = plsc.VectorSubcoreMesh(core_axis_name="c", subcore_axis_name="s",
                                  num_cores=NC, num_subcores=NS)
    wflat_sc, part, cnt_sc = pl.kernel(
        _sc_body,
        out_type=[
            jax.ShapeDtypeStruct((S_SC,), jnp.float32),
            jax.ShapeDtypeStruct((NW, B * D), jnp.float32),
            jax.ShapeDtypeStruct((NW, L), jnp.float32),
        ],
        mesh=mesh,
        compiler_params=pltpu.CompilerParams(needs_layout_passes=False),
        scratch_types=[
            pltpu.VMEM((CH, D), jnp.float32),
            pltpu.VMEM((CH, D), jnp.float32),
            pltpu.VMEM((ROWS_W,), jnp.int32),
            pltpu.VMEM((D,), jnp.float32),
            pltpu.VMEM((L,), jnp.float32),
            pltpu.VMEM((ROWS_W,), jnp.float32),
            pltpu.VMEM((B * D,), jnp.float32),
            pltpu.VMEM((L,), jnp.float32),
            pltpu.SemaphoreType.DMA,
            pltpu.SemaphoreType.DMA,
        ],
    )(x, batch, wf, bvec)

    pooled_tc, weights_tc, cnt_tc = pl.pallas_call(
        _tc_body,
        grid=(S_TC // BLOCK_R,),
        in_specs=[
            pl.BlockSpec((BLOCK_R, D), lambda i: (i, 0)),
            pl.BlockSpec((BLOCK_R, 1), lambda i: (i, 0)),
            pl.BlockSpec((D, 1), lambda i: (0, 0)),
            pl.BlockSpec((1, 1), lambda i: (0, 0)),
        ],
        out_specs=[
            pl.BlockSpec((B, D), lambda i: (0, 0)),
            pl.BlockSpec((BLOCK_R, 1), lambda i: (i, 0)),
            pl.BlockSpec((1, 1), lambda i: (0, 0)),
        ],
        out_shape=[
            jax.ShapeDtypeStruct((B, D), jnp.float32),
            jax.ShapeDtypeStruct((S_TC, 1), jnp.float32),
            jax.ShapeDtypeStruct((1, 1), jnp.float32),
        ],
        compiler_params=pltpu.CompilerParams(
            dimension_semantics=("arbitrary",),
        ),
    )(x, batch2, W, b2)

    pooled, ratio = pl.pallas_call(
        _combine_body,
        out_shape=[
            jax.ShapeDtypeStruct((B, D), jnp.float32),
            jax.ShapeDtypeStruct((1, 1), jnp.float32),
        ],
    )(part.reshape(NW, B, D), cnt_sc, pooled_tc, cnt_tc)

    weights = jnp.concatenate([weights_tc, wflat_sc.reshape(S_SC, 1)], axis=0)
    return pooled, weights, ratio.reshape(())
